# baseline (speedup 1.0000x reference)
import functools

import jax
import jax.numpy as jnp
from jax import lax
from jax.experimental import pallas as pl
from jax.experimental.pallas import tpu as pltpu
from jax.experimental.pallas import tpu_sc as plsc

_W = 16
_PB = 256
_NB = 400


def _prep_body(nbins, d, f, beta_col, x0_ref, v_ref, beta_ref, tpos_ref):
    nb = x0_ref.shape[0]
    pieces = [x0_ref[...]]
    for b in range(nbins):
        pieces.append(v_ref[b])
    pieces.append(beta_ref[...])
    pieces.append(jnp.zeros((nb, f - beta_col - 1), jnp.float32))
    tpos_ref[...] = jnp.concatenate(pieces, axis=1)


def _prep_tables(x0, v, beta, nbins, d, f, beta_col):
    n = x0.shape[0]
    body = functools.partial(_prep_body, nbins, d, f, beta_col)
    return pl.pallas_call(
        body,
        grid=(n // _NB,),
        in_specs=[
            pl.BlockSpec((_NB, d), lambda i: (i, 0)),
            pl.BlockSpec((nbins, _NB, d), lambda i: (0, i, 0)),
            pl.BlockSpec((_NB, 1), lambda i: (i, 0)),
        ],
        out_specs=pl.BlockSpec((_NB, f), lambda i: (i, 0)),
        out_shape=jax.ShapeDtypeStruct((n, f), jnp.float32),
    )(x0, v, beta[:, None])


def _sc_gather(table, idx_i, idx_j, ppad, f):
    mesh = plsc.VectorSubcoreMesh(core_axis_name="c", subcore_axis_name="s")
    n_workers = mesh.num_cores * mesh.num_subcores
    bpw = ppad // n_workers
    nch = bpw // _W

    out_t = (
        jax.ShapeDtypeStruct((ppad, f), jnp.float32),
        jax.ShapeDtypeStruct((ppad, f), jnp.float32),
    )

    @functools.partial(
        pl.kernel, out_type=out_t, mesh=mesh,
        scratch_types=[
            pltpu.VMEM((bpw,), jnp.int32),
            pltpu.VMEM((bpw,), jnp.int32),
            pltpu.VMEM((_W, f), jnp.float32),
            pltpu.VMEM((_W, f), jnp.float32),
            pltpu.VMEM((_W, f), jnp.float32),
            pltpu.VMEM((_W, f), jnp.float32),
            pltpu.SemaphoreType.DMA,
            pltpu.SemaphoreType.DMA,
            pltpu.SemaphoreType.DMA,
            pltpu.SemaphoreType.DMA,
        ])
    def gather_kernel(table_hbm, ii_hbm, ij_hbm, gi_hbm, gj_hbm,
                      ii_v, ij_v, bi0, bj0, bi1, bj1, si0, sj0, si1, sj1):
        wid = lax.axis_index("s") * mesh.num_cores + lax.axis_index("c")
        base = wid * bpw
        pltpu.sync_copy(ii_hbm.at[pl.ds(base, bpw)], ii_v)
        pltpu.sync_copy(ij_hbm.at[pl.ds(base, bpw)], ij_v)

        def fire(c, bi, bj, si, sj):
            pltpu.make_async_copy(
                table_hbm.at[ii_v.at[pl.ds(c * _W, _W)]], bi, si).start()
            pltpu.make_async_copy(
                table_hbm.at[ij_v.at[pl.ds(c * _W, _W)]], bj, sj).start()

        def drain(c, bi, bj, si, sj):
            pltpu.make_async_copy(
                table_hbm.at[ii_v.at[pl.ds(c * _W, _W)]], bi, si).wait()
            pltpu.make_async_copy(
                table_hbm.at[ij_v.at[pl.ds(c * _W, _W)]], bj, sj).wait()
            pltpu.sync_copy(bi, gi_hbm.at[pl.ds(base + c * _W, _W)])
            pltpu.sync_copy(bj, gj_hbm.at[pl.ds(base + c * _W, _W)])

        fire(0, bi0, bj0, si0, sj0)

        @pl.loop(0, nch, step=2)
        def _(c):
            fire(c + 1, bi1, bj1, si1, sj1)
            drain(c, bi0, bj0, si0, sj0)

            @pl.when(c + 2 < nch)
            def _():
                fire(c + 2, bi0, bj0, si0, sj0)

            drain(c + 1, bi1, bj1, si1, sj1)

    return gather_kernel(table, idx_i, idx_j)


def _tc_body(nbins, d, t_len, beta_col, times_ref, gi_ref, gj_ref,
             out_ref, abc_ref):
    gi = gi_ref[...]
    gj = gj_ref[...]
    lane = lax.broadcasted_iota(jnp.int32, gi.shape, 1)
    x = jnp.where(lane == beta_col, gi + gj, gi - gj)
    xt = jnp.transpose(x)

    inv_w = jnp.float32(1.0 / nbins)
    acc = xt[0:d, :]
    for m in range(nbins):
        dvm = xt[d + d * m:d + d * (m + 1), :]
        abc_ref[m:m + 1, :] = jnp.sum(acc * acc, axis=0, keepdims=True)
        abc_ref[nbins + m:nbins + m + 1, :] = jnp.sum(
            acc * dvm, axis=0, keepdims=True)
        abc_ref[2 * nbins + m:2 * nbins + m + 1, :] = jnp.sum(
            dvm * dvm, axis=0, keepdims=True)
        acc = acc + dvm * inv_w

    t = times_ref[...]
    mt = jnp.clip(jnp.floor(t * nbins), 0.0, nbins - 1.0)
    r = t - mt * inv_w
    lane2 = lax.broadcasted_iota(jnp.int32, (t_len, 3 * nbins), 1)
    binl = (lane2 % nbins).astype(jnp.float32)
    coef = jnp.where(lane2 < nbins, jnp.float32(1.0),
                     jnp.where(lane2 < 2 * nbins, 2.0 * r, r * r))
    sel = jnp.where(binl == mt, coef, jnp.float32(0.0))

    norm2 = lax.dot_general(
        sel, abc_ref[...], (((1,), (0,)), ((), ())),
        preferred_element_type=jnp.float32,
        precision=lax.Precision.HIGHEST)
    bsum = xt[beta_col:beta_col + 1, :]
    out_ref[...] = jnp.exp(bsum - norm2)


def _tc_compute(times2d, gi, gj, nbins, d, f, beta_col, ppad):
    t_len = times2d.shape[0]
    body = functools.partial(_tc_body, nbins, d, t_len, beta_col)
    return pl.pallas_call(
        body,
        grid=(ppad // _PB,),
        in_specs=[
            pl.BlockSpec((t_len, 1), lambda p: (0, 0)),
            pl.BlockSpec((_PB, f), lambda p: (p, 0)),
            pl.BlockSpec((_PB, f), lambda p: (p, 0)),
        ],
        out_specs=pl.BlockSpec((t_len, _PB), lambda p: (0, p)),
        out_shape=jax.ShapeDtypeStruct((t_len, ppad), jnp.float32),
        scratch_shapes=[pltpu.VMEM((3 * nbins, _PB), jnp.float32)],
    )(times2d, gi, gj)


def kernel(x0, v, beta, times_list, node_pairs):
    n, d = x0.shape
    nbins = v.shape[0]
    p = node_pairs.shape[1]

    beta_col = d + nbins * d
    f = ((beta_col + 1 + 127) // 128) * 128
    table = _prep_tables(x0, v, beta, nbins, d, f, beta_col)

    align = max(_W * 64, _PB)
    ppad = ((p + align - 1) // align) * align
    idx = jnp.pad(node_pairs, ((0, 0), (0, ppad - p)))
    gi, gj = _sc_gather(table, idx[0], idx[1], ppad, f)

    out = _tc_compute(times_list[:, None], gi, gj, nbins, d, f, beta_col,
                      ppad)
    return out[:, :p]


# speedup vs baseline: 1.4773x; 1.4773x over previous
import functools

import jax
import jax.numpy as jnp
from jax import lax
from jax.experimental import pallas as pl
from jax.experimental.pallas import tpu as pltpu
from jax.experimental.pallas import tpu_sc as plsc

_W = 16
_PB = 256


def _sc_gather(table, idx_i, idx_j, ppad, f):
    mesh = plsc.VectorSubcoreMesh(core_axis_name="c", subcore_axis_name="s")
    n_workers = mesh.num_cores * mesh.num_subcores
    bpw = ppad // n_workers
    nch = bpw // _W

    out_t = (
        jax.ShapeDtypeStruct((ppad, f), jnp.float32),
        jax.ShapeDtypeStruct((ppad, f), jnp.float32),
    )

    @functools.partial(
        pl.kernel, out_type=out_t, mesh=mesh,
        scratch_types=[
            pltpu.VMEM((bpw,), jnp.int32),
            pltpu.VMEM((bpw,), jnp.int32),
            pltpu.VMEM((_W, f), jnp.float32),
            pltpu.VMEM((_W, f), jnp.float32),
            pltpu.VMEM((_W, f), jnp.float32),
            pltpu.VMEM((_W, f), jnp.float32),
            pltpu.SemaphoreType.DMA,
            pltpu.SemaphoreType.DMA,
            pltpu.SemaphoreType.DMA,
            pltpu.SemaphoreType.DMA,
        ])
    def gather_kernel(table_hbm, ii_hbm, ij_hbm, gi_hbm, gj_hbm,
                      ii_v, ij_v, bi0, bj0, bi1, bj1, si0, sj0, si1, sj1):
        wid = lax.axis_index("s") * mesh.num_cores + lax.axis_index("c")
        base = wid * bpw
        pltpu.sync_copy(ii_hbm.at[pl.ds(base, bpw)], ii_v)
        pltpu.sync_copy(ij_hbm.at[pl.ds(base, bpw)], ij_v)

        def fire(c, bi, bj, si, sj):
            pltpu.make_async_copy(
                table_hbm.at[ii_v.at[pl.ds(c * _W, _W)]], bi, si).start()
            pltpu.make_async_copy(
                table_hbm.at[ij_v.at[pl.ds(c * _W, _W)]], bj, sj).start()

        def drain(c, bi, bj, si, sj):
            pltpu.make_async_copy(
                table_hbm.at[ii_v.at[pl.ds(c * _W, _W)]], bi, si).wait()
            pltpu.make_async_copy(
                table_hbm.at[ij_v.at[pl.ds(c * _W, _W)]], bj, sj).wait()
            pltpu.sync_copy(bi, gi_hbm.at[pl.ds(base + c * _W, _W)])
            pltpu.sync_copy(bj, gj_hbm.at[pl.ds(base + c * _W, _W)])

        fire(0, bi0, bj0, si0, sj0)

        @pl.loop(0, nch, step=2)
        def _(c):
            fire(c + 1, bi1, bj1, si1, sj1)
            drain(c, bi0, bj0, si0, sj0)

            @pl.when(c + 2 < nch)
            def _():
                fire(c + 2, bi0, bj0, si0, sj0)

            drain(c + 1, bi1, bj1, si1, sj1)

    return gather_kernel(table, idx_i, idx_j)


def _tc_body(nbins, d, t_len, beta_col, times_ref, gi_ref, gj_ref,
             out_ref, abc_ref):
    gi = gi_ref[...]
    gj = gj_ref[...]
    lane = lax.broadcasted_iota(jnp.int32, gi.shape, 1)
    x = jnp.where(lane == beta_col, gi + gj, gi - gj)
    xt = jnp.transpose(x)

    inv_w = jnp.float32(1.0 / nbins)
    acc = xt[0:d, :]
    for m in range(nbins):
        dvm = xt[d + d * m:d + d * (m + 1), :]
        abc_ref[nbins + m:nbins + m + 1, :] = jnp.sum(
            acc * dvm, axis=0, keepdims=True)
        abc_ref[2 * nbins + m:2 * nbins + m + 1, :] = jnp.sum(
            dvm * dvm, axis=0, keepdims=True)
        acc = acc + dvm * inv_w
    b_all = abc_ref[nbins:2 * nbins, :]
    c_all = abc_ref[2 * nbins:3 * nbins, :]
    dx0 = xt[0:d, :]
    a0 = jnp.sum(dx0 * dx0, axis=0, keepdims=True)
    step = 2.0 * inv_w * b_all + (inv_w * inv_w) * c_all
    row = lax.broadcasted_iota(jnp.int32, (nbins, nbins), 0)
    col = lax.broadcasted_iota(jnp.int32, (nbins, nbins), 1)
    ltri = jnp.where(col < row, jnp.float32(1.0), jnp.float32(0.0))
    a_all = a0 + lax.dot_general(
        ltri, step, (((1,), (0,)), ((), ())),
        preferred_element_type=jnp.float32,
        precision=lax.Precision.HIGHEST)
    abc_ref[0:nbins, :] = a_all

    t = times_ref[...]
    mt = jnp.clip(jnp.floor(t * nbins), 0.0, nbins - 1.0)
    r = t - mt * inv_w
    lane2 = lax.broadcasted_iota(jnp.int32, (t_len, 3 * nbins), 1)
    binl = (lane2 % nbins).astype(jnp.float32)
    coef = jnp.where(lane2 < nbins, jnp.float32(1.0),
                     jnp.where(lane2 < 2 * nbins, 2.0 * r, r * r))
    sel = jnp.where(binl == mt, coef, jnp.float32(0.0))

    norm2 = lax.dot_general(
        sel, abc_ref[...], (((1,), (0,)), ((), ())),
        preferred_element_type=jnp.float32,
        precision=lax.Precision.HIGHEST)
    bsum = xt[beta_col:beta_col + 1, :]
    out_ref[...] = jnp.exp(bsum - norm2)


def _tc_compute(times2d, gi, gj, nbins, d, f, beta_col, ppad):
    t_len = times2d.shape[0]
    body = functools.partial(_tc_body, nbins, d, t_len, beta_col)
    return pl.pallas_call(
        body,
        grid=(ppad // _PB,),
        in_specs=[
            pl.BlockSpec((t_len, 1), lambda p: (0, 0)),
            pl.BlockSpec((_PB, f), lambda p: (p, 0)),
            pl.BlockSpec((_PB, f), lambda p: (p, 0)),
        ],
        out_specs=pl.BlockSpec((t_len, _PB), lambda p: (0, p)),
        out_shape=jax.ShapeDtypeStruct((t_len, ppad), jnp.float32),
        scratch_shapes=[pltpu.VMEM((3 * nbins, _PB), jnp.float32)],
    )(times2d, gi, gj)


def kernel(x0, v, beta, times_list, node_pairs):
    n, d = x0.shape
    nbins = v.shape[0]
    p = node_pairs.shape[1]

    beta_col = d + nbins * d
    f = ((beta_col + 1 + 127) // 128) * 128
    vt = jnp.transpose(v, (1, 0, 2)).reshape(n, nbins * d)
    table = jnp.concatenate(
        [x0, vt, beta[:, None],
         jnp.zeros((n, f - beta_col - 1), jnp.float32)], axis=1)

    align = max(_W * 64, _PB)
    ppad = ((p + align - 1) // align) * align
    idx = jnp.pad(node_pairs, ((0, 0), (0, ppad - p)))
    gi, gj = _sc_gather(table, idx[0], idx[1], ppad, f)

    out = _tc_compute(times_list[:, None], gi, gj, nbins, d, f, beta_col,
                      ppad)
    return out[:, :p]


# speedup vs baseline: 1.5760x; 1.0668x over previous
import functools

import jax
import jax.numpy as jnp
from jax import lax
from jax.experimental import pallas as pl
from jax.experimental.pallas import tpu as pltpu
from jax.experimental.pallas import tpu_sc as plsc

_W = 16
_PB = 256


def _sc_gather(table, idx_i, idx_j, ppad, f):
    mesh = plsc.VectorSubcoreMesh(core_axis_name="c", subcore_axis_name="s")
    n_workers = mesh.num_cores * mesh.num_subcores
    bpw = ppad // n_workers
    nch = bpw // _W

    out_t = (
        jax.ShapeDtypeStruct((ppad, f), jnp.float32),
        jax.ShapeDtypeStruct((ppad, f), jnp.float32),
    )

    @functools.partial(
        pl.kernel, out_type=out_t, mesh=mesh,
        scratch_types=[
            pltpu.VMEM((bpw,), jnp.int32),
            pltpu.VMEM((bpw,), jnp.int32),
            pltpu.VMEM((_W, f), jnp.float32),
            pltpu.VMEM((_W, f), jnp.float32),
            pltpu.VMEM((_W, f), jnp.float32),
            pltpu.VMEM((_W, f), jnp.float32),
            pltpu.SemaphoreType.DMA,
            pltpu.SemaphoreType.DMA,
            pltpu.SemaphoreType.DMA,
            pltpu.SemaphoreType.DMA,
        ])
    def gather_kernel(table_hbm, ii_hbm, ij_hbm, gi_hbm, gj_hbm,
                      ii_v, ij_v, bi0, bj0, bi1, bj1, si0, sj0, si1, sj1):
        wid = lax.axis_index("s") * mesh.num_cores + lax.axis_index("c")
        base = wid * bpw
        pltpu.sync_copy(ii_hbm.at[pl.ds(base, bpw)], ii_v)
        pltpu.sync_copy(ij_hbm.at[pl.ds(base, bpw)], ij_v)

        def fire(c, bi, bj, si, sj):
            pltpu.make_async_copy(
                table_hbm.at[ii_v.at[pl.ds(c * _W, _W)]], bi, si).start()
            pltpu.make_async_copy(
                table_hbm.at[ij_v.at[pl.ds(c * _W, _W)]], bj, sj).start()

        def drain(c, bi, bj, si, sj):
            pltpu.make_async_copy(
                table_hbm.at[ii_v.at[pl.ds(c * _W, _W)]], bi, si).wait()
            pltpu.make_async_copy(
                table_hbm.at[ij_v.at[pl.ds(c * _W, _W)]], bj, sj).wait()
            pltpu.sync_copy(bi, gi_hbm.at[pl.ds(base + c * _W, _W)])
            pltpu.sync_copy(bj, gj_hbm.at[pl.ds(base + c * _W, _W)])

        fire(0, bi0, bj0, si0, sj0)

        @pl.loop(0, nch, step=2)
        def _(c):
            fire(c + 1, bi1, bj1, si1, sj1)
            drain(c, bi0, bj0, si0, sj0)

            @pl.when(c + 2 < nch)
            def _():
                fire(c + 2, bi0, bj0, si0, sj0)

            drain(c + 1, bi1, bj1, si1, sj1)

    return gather_kernel(table, idx_i, idx_j)


def _tc_body(nbins, d, t_len, beta_col, times_ref, gi_ref, gj_ref,
             out_ref, abc_ref):
    gi = gi_ref[...]
    gj = gj_ref[...]
    lane = lax.broadcasted_iota(jnp.int32, gi.shape, 1)
    x = jnp.where(lane == beta_col, gi + gj, gi - gj)
    xt = jnp.transpose(x)

    inv_w = jnp.float32(1.0 / nbins)
    acc = xt[0:d, :]
    for m in range(nbins):
        dvm = xt[d + d * m:d + d * (m + 1), :]
        abc_ref[nbins + m:nbins + m + 1, :] = jnp.sum(
            acc * dvm, axis=0, keepdims=True)
        abc_ref[2 * nbins + m:2 * nbins + m + 1, :] = jnp.sum(
            dvm * dvm, axis=0, keepdims=True)
        acc = acc + dvm * inv_w
    b_all = abc_ref[nbins:2 * nbins, :]
    c_all = abc_ref[2 * nbins:3 * nbins, :]
    dx0 = xt[0:d, :]
    a0 = jnp.sum(dx0 * dx0, axis=0, keepdims=True)
    step = 2.0 * inv_w * b_all + (inv_w * inv_w) * c_all
    row = lax.broadcasted_iota(jnp.int32, (nbins, nbins), 0)
    col = lax.broadcasted_iota(jnp.int32, (nbins, nbins), 1)
    ltri = jnp.where(col < row, jnp.float32(1.0), jnp.float32(0.0))
    a_all = a0 + lax.dot_general(
        ltri, step, (((1,), (0,)), ((), ())),
        preferred_element_type=jnp.float32,
        precision=lax.Precision.HIGHEST)
    abc_ref[0:nbins, :] = a_all

    t = times_ref[...]
    mt = jnp.clip(jnp.floor(t * nbins), 0.0, nbins - 1.0)
    r = t - mt * inv_w
    lane2 = lax.broadcasted_iota(jnp.int32, (t_len, 3 * nbins), 1)
    binl = (lane2 % nbins).astype(jnp.float32)
    coef = jnp.where(lane2 < nbins, jnp.float32(1.0),
                     jnp.where(lane2 < 2 * nbins, 2.0 * r, r * r))
    sel = jnp.where(binl == mt, coef, jnp.float32(0.0))

    norm2 = lax.dot_general(
        sel, abc_ref[...], (((1,), (0,)), ((), ())),
        preferred_element_type=jnp.float32,
        precision=lax.Precision.HIGHEST)
    bsum = xt[beta_col:beta_col + 1, :]
    out_ref[...] = jnp.exp(bsum - norm2)


def _tc_compute(times2d, gi, gj, nbins, d, f, beta_col, ppad):
    t_len = times2d.shape[0]
    body = functools.partial(_tc_body, nbins, d, t_len, beta_col)
    return pl.pallas_call(
        body,
        grid=(ppad // _PB,),
        in_specs=[
            pl.BlockSpec((t_len, 1), lambda p: (0, 0)),
            pl.BlockSpec((_PB, f), lambda p: (p, 0)),
            pl.BlockSpec((_PB, f), lambda p: (p, 0)),
        ],
        out_specs=pl.BlockSpec((t_len, _PB), lambda p: (0, p)),
        out_shape=jax.ShapeDtypeStruct((t_len, ppad), jnp.float32),
        scratch_shapes=[pltpu.VMEM((3 * nbins, _PB), jnp.float32)],
    )(times2d, gi, gj)


def kernel(x0, v, beta, times_list, node_pairs):
    n, d = x0.shape
    nbins = v.shape[0]
    p = node_pairs.shape[1]

    beta_col = d + nbins * d
    f = ((beta_col + 1 + 127) // 128) * 128
    vt = jnp.transpose(v, (1, 0, 2)).reshape(n, nbins * d)
    table = jnp.concatenate(
        [x0, vt, beta[:, None],
         jnp.zeros((n, f - beta_col - 1), jnp.float32)], axis=1)

    nslice = 4
    align = nslice * max(_W * 64, _PB)
    ppad = ((p + align - 1) // align) * align
    idx = jnp.pad(node_pairs, ((0, 0), (0, ppad - p)))

    psl = ppad // nslice
    times2d = times_list[:, None]
    outs = []
    for s in range(nslice):
        ii = lax.dynamic_slice_in_dim(idx[0], s * psl, psl)
        ij = lax.dynamic_slice_in_dim(idx[1], s * psl, psl)
        gi, gj = _sc_gather(table, ii, ij, psl, f)
        outs.append(
            _tc_compute(times2d, gi, gj, nbins, d, f, beta_col, psl))
    out = jnp.concatenate(outs, axis=1)
    return out[:, :p]
